# baseline (speedup 1.0000x reference)
import jax
import jax.numpy as jnp
from jax import lax
from jax.experimental import pallas as pl
from jax.experimental.pallas import tpu as pltpu
from jax.experimental.pallas import tpu_sc as plsc

MAX_NUM_PAGES = 2048
PAGE_SIZE = 16
N_HEADS = 8
HEAD_DIM = 128
T = 128
NROWS = MAX_NUM_PAGES * 2 * PAGE_SIZE
L = 16
NVREG = T // L
NWORKERS = 32
STRIPE = NROWS // NWORKERS
NCHUNK = 4
CHUNK = STRIPE // NCHUNK


def _sc_body(k_hbm, v_hbm, a_lo_hbm, a_hi_hbm, p_lo_hbm, p_hi_hbm,
             ll_hbm, pidx_hbm, cache_hbm, out_hbm,
             a_lo_v, a_hi_v, p_lo_v, p_hi_v, ll_v, pidx_v,
             copy_sem, tok_sem):
  wid = lax.axis_index("s") * 2 + lax.axis_index("c")
  base_r = wid * STRIPE

  copies = []
  for c in range(NCHUNK):
    r0 = base_r + c * CHUNK
    copies.append(pltpu.async_copy(
        cache_hbm.at[pl.ds(r0, CHUNK)], out_hbm.at[pl.ds(r0, CHUNK)],
        copy_sem))

  pltpu.sync_copy(a_lo_hbm, a_lo_v)
  pltpu.sync_copy(a_hi_hbm, a_hi_v)
  pltpu.sync_copy(p_lo_hbm, p_lo_v)
  pltpu.sync_copy(p_hi_hbm, p_hi_v)
  pltpu.sync_copy(ll_hbm, ll_v)
  pltpu.sync_copy(pidx_hbm, pidx_v)

  dest_regs = []
  for g in range(NVREG):
    t = lax.iota(jnp.int32, L) + (g * L)
    a_lo = a_lo_v[pl.ds(g * L, L)]
    a_hi = a_hi_v[pl.ds(g * L, L)]
    p_lo = p_lo_v[pl.ds(g * L, L)]
    p_hi = p_hi_v[pl.ds(g * L, L)]
    ll = ll_v[pl.ds(g * L, L)]
    j = t - a_lo
    append_len = a_hi - a_lo
    n_pages = p_hi - p_lo
    kv_len = (n_pages - 1) * PAGE_SIZE + ll
    pos = kv_len - append_len + j
    page_local = lax.shift_right_arithmetic(pos, 4)
    slot = lax.bitwise_and(pos, PAGE_SIZE - 1)
    page = plsc.load_gather(pidx_v, [p_lo + page_local])
    dest_regs.append(page * (2 * PAGE_SIZE) + slot)

  for c in copies:
    c.wait()

  for u in range(T):
    d = dest_regs[u // L][u % L]

    @pl.when(jnp.logical_and(d >= base_r, d < base_r + STRIPE))
    def _write(u=u, d=d):
      pltpu.async_copy(k_hbm.at[u], out_hbm.at[d], tok_sem).wait()
      pltpu.async_copy(v_hbm.at[u], out_hbm.at[d + PAGE_SIZE], tok_sem).wait()


_sc_append = pl.kernel(
    _sc_body,
    out_type=jax.ShapeDtypeStruct((NROWS, N_HEADS, HEAD_DIM), jnp.float32),
    mesh=plsc.VectorSubcoreMesh(core_axis_name="c", subcore_axis_name="s"),
    compiler_params=pltpu.CompilerParams(needs_layout_passes=False),
    scratch_types=[
        pltpu.VMEM((T,), jnp.int32),
        pltpu.VMEM((T,), jnp.int32),
        pltpu.VMEM((T,), jnp.int32),
        pltpu.VMEM((T,), jnp.int32),
        pltpu.VMEM((T,), jnp.int32),
        pltpu.VMEM((T,), jnp.int32),
        pltpu.SemaphoreType.DMA,
        pltpu.SemaphoreType.DMA,
    ],
    name="paged_kv_append",
)


def kernel(k, v, kv_append_indptr, kv_cache, kv_page_indices, kv_page_indptr,
           kv_page_lastlen):
  a_lo = kv_append_indptr[:T]
  a_hi = kv_append_indptr[1:T + 1]
  p_lo = kv_page_indptr[:T]
  p_hi = kv_page_indptr[1:T + 1]
  out = _sc_append(k, v, a_lo, a_hi, p_lo, p_hi, kv_page_lastlen,
                   kv_page_indices, kv_cache.reshape(NROWS, N_HEADS, HEAD_DIM))
  return out.reshape(kv_cache.shape)


# speedup vs baseline: 41.2650x; 41.2650x over previous
import jax
import jax.numpy as jnp
from jax import lax
from jax.experimental import pallas as pl
from jax.experimental.pallas import tpu as pltpu

MAX_NUM_PAGES = 2048
PAGE_SIZE = 16
N_HEADS = 8
HEAD_DIM = 128
T = 128
NROWS = MAX_NUM_PAGES * 2 * PAGE_SIZE
BLK = 1024
NBLK = NROWS // BLK


def _tc_body(a_lo_s, a_hi_s, p_lo_s, p_hi_s, ll_s, pidx_s,
             cache_blk, k_ref, v_ref, out_blk, dest_s):
  i = pl.program_id(0)

  @pl.when(i == 0)
  def _prep():
    def body(u, _):
      a_lo = a_lo_s[u]
      append_len = a_hi_s[u] - a_lo
      n_pages = p_hi_s[u] - p_lo_s[u]
      pos = (n_pages - 1) * PAGE_SIZE + ll_s[u] - append_len + (u - a_lo)
      page = pidx_s[p_lo_s[u] + (pos // PAGE_SIZE)]
      dest_s[u] = page * (2 * PAGE_SIZE) + lax.rem(pos, PAGE_SIZE)
      return 0
    lax.fori_loop(0, T, body, 0)

  out_blk[...] = cache_blk[...]

  base = i * BLK

  def tok(u, _):
    d = dest_s[u]

    @pl.when(jnp.logical_and(d >= base, d < base + BLK))
    def _w():
      loc = d - base
      out_blk[pl.ds(loc, 1)] = k_ref[pl.ds(u, 1)]
      out_blk[pl.ds(loc + PAGE_SIZE, 1)] = v_ref[pl.ds(u, 1)]
    return 0

  lax.fori_loop(0, T, tok, 0)


_tc_append = pl.pallas_call(
    _tc_body,
    grid_spec=pltpu.PrefetchScalarGridSpec(
        num_scalar_prefetch=6,
        grid=(NBLK,),
        in_specs=[
            pl.BlockSpec((BLK, N_HEADS, HEAD_DIM), lambda i, *_: (i, 0, 0)),
            pl.BlockSpec((T, N_HEADS, HEAD_DIM), lambda i, *_: (0, 0, 0)),
            pl.BlockSpec((T, N_HEADS, HEAD_DIM), lambda i, *_: (0, 0, 0)),
        ],
        out_specs=pl.BlockSpec((BLK, N_HEADS, HEAD_DIM),
                               lambda i, *_: (i, 0, 0)),
        scratch_shapes=[pltpu.SMEM((T,), jnp.int32)],
    ),
    out_shape=jax.ShapeDtypeStruct((NROWS, N_HEADS, HEAD_DIM), jnp.float32),
    compiler_params=pltpu.CompilerParams(
        dimension_semantics=("arbitrary",)),
)


def kernel(k, v, kv_append_indptr, kv_cache, kv_page_indices, kv_page_indptr,
           kv_page_lastlen):
  a_lo = kv_append_indptr[:T]
  a_hi = kv_append_indptr[1:T + 1]
  p_lo = kv_page_indptr[:T]
  p_hi = kv_page_indptr[1:T + 1]
  out = _tc_append(a_lo, a_hi, p_lo, p_hi, kv_page_lastlen, kv_page_indices,
                   kv_cache.reshape(NROWS, N_HEADS, HEAD_DIM), k, v)
  return out.reshape(kv_cache.shape)
